# baseline (speedup 1.0000x reference)
import functools

import jax
import jax.numpy as jnp
from jax import lax
from jax.experimental import pallas as pl
from jax.experimental.pallas import tpu as pltpu
from jax.experimental.pallas import tpu_sc as plsc

_B = 4
_N = 262144
_SAMPLE = 16384
_OUTP = _SAMPLE + 16
_NVOX = 512000
_SENT = 524288
_EMPTY = 0x7FFFFFFF
_TBL = 32768
_NSUB = 16
_NCORE = 2
_CHUNK = 2048
_LSEG = 16384
_PCHUNK = _N // _NSUB
_SCH = 128


def _bin_body(xyz_ref, lin_ref):
    lower = jnp.float32(-4.0)
    voxel = jnp.float32(0.1)
    grid = jnp.float32(80.0)
    x = xyz_ref[0, 0, :]
    y = xyz_ref[0, 1, :]
    z = xyz_ref[0, 2, :]
    cx = jnp.floor((x - lower) / voxel)
    cy = jnp.floor((y - lower) / voxel)
    cz = jnp.floor((z - lower) / voxel)
    valid = (
        (cx >= 0.0) & (cx < grid)
        & (cy >= 0.0) & (cy < grid)
        & (cz >= 0.0) & (cz < grid)
    )
    zero = jnp.float32(0.0)
    ix = jnp.where(valid, cx, zero).astype(jnp.int32)
    iy = jnp.where(valid, cy, zero).astype(jnp.int32)
    iz = jnp.where(valid, cz, zero).astype(jnp.int32)
    lin = ix * 6400 + iy * 80 + iz
    lin_ref[0, 0, :] = jnp.where(valid, lin, jnp.int32(_SENT))


def _voxel_ids(xyz_t):
    return pl.pallas_call(
        _bin_body,
        out_shape=jax.ShapeDtypeStruct((_B, 1, _N), jnp.int32),
        grid=(_B,),
        in_specs=[pl.BlockSpec((1, 3, _N), lambda i: (i, 0, 0))],
        out_specs=pl.BlockSpec((1, 1, _N), lambda i: (i, 0, 0)),
    )(xyz_t).reshape(_B, _N)


def _sc_body(
    lin_hbm,
    xyzf_hbm,
    out_hbm,
    tbl_v,
    buf_v,
    lst_v,
    sidx_v,
    pres_v,
    cmp_v,
    zbuf_v,
    ones_v,
    gidx_v,
    gx_v,
    gy_v,
    gz_v,
    ox_v,
    oy_v,
    oz_v,
    cnt_v,
    cnts_v,
    lin_s,
    pres_s,
    cnts_s,
    sem_g,
    sem_s,
):
    sid = lax.axis_index("s")
    cid = lax.axis_index("c")
    iota = lax.iota(jnp.int32, 16)
    my_tid = jnp.int32(sid)
    prev_idx = jnp.maximum(iota - 1, 0)

    @functools.partial(lax.fori_loop, 0, 1024 // 16, init_val=0)
    def _(i, c):
        zbuf_v[pl.ds(i * 16, 16)] = jnp.zeros((16,), jnp.int32)
        return c

    @functools.partial(lax.fori_loop, 0, _SCH // 16, init_val=0)
    def _(i, c):
        ones_v[pl.ds(i * 16, 16)] = jnp.ones((16,), jnp.int32)
        return c

    def one_batch(step, _carry):
        b = cid * 2 + step

        for j in range(_PCHUNK // 1024):
            pltpu.sync_copy(
                zbuf_v, pres_s.at[pl.ds(sid * _PCHUNK + j * 1024, 1024)]
            )
        pltpu.sync_copy(
            lin_hbm.at[b, pl.ds(sid * _PCHUNK, _PCHUNK)],
            lin_s.at[pl.ds(sid * _PCHUNK, _PCHUNK)],
        )
        @functools.partial(lax.fori_loop, 0, _TBL // 16, init_val=0, unroll=4)
        def _(i, c):
            tbl_v[pl.ds(i * 16, 16)] = jnp.full((16,), _EMPTY, jnp.int32)
            return c

        plsc.subcore_barrier()

        def chunk_body(k, _c):
            pltpu.sync_copy(lin_s.at[pl.ds(k * _CHUNK, _CHUNK)], buf_v)
            base = k * _CHUNK

            @functools.partial(
                lax.fori_loop, 0, _CHUNK // 16, init_val=0, unroll=4
            )
            def _(v, c):
                lin = buf_v[pl.ds(v * 16, 16)]
                own = lax.shift_right_logical(lin, 15) == my_tid
                lidx = lin & jnp.int32(_TBL - 1)
                ivec = base + v * 16 + iota
                cur = plsc.load_gather(tbl_v, [lidx], mask=own)
                pending = own & (ivec < cur)

                def fix_body(p):
                    plsc.store_scatter(tbl_v, [lidx], ivec, mask=p)
                    got = plsc.load_gather(tbl_v, [lidx], mask=p)
                    return p & (ivec < got)

                lax.while_loop(
                    lambda p: jnp.any(p), fix_body, pending
                )
                return c

            return 0

        lax.fori_loop(0, _N // _CHUNK, chunk_body, 0)

        def half_seg(seg, _c0):
            def scan_tbl(v, off):
                vals = tbl_v[pl.ds(seg * _LSEG + v * 16, 16)]
                sel = vals != _EMPTY
                plsc.store_compressed(
                    lst_v.at[pl.ds(off, 16)], vals, mask=sel
                )
                return off + jnp.sum(sel.astype(jnp.int32))

            cnt_own = lax.fori_loop(0, _LSEG // 16, scan_tbl, jnp.int32(0))

            @functools.partial(lax.fori_loop, 0, _SCH // 16, init_val=0)
            def _(i, c):
                lst_v[pl.ds(cnt_own + i * 16, 16)] = jnp.full(
                    (16,), _N, jnp.int32
                )
                return c

            def add_chunk(k, _c):
                for t in range(_SCH // 16):
                    sidx_v[pl.ds(t * 16, 16)] = lst_v[
                        pl.ds(k * _SCH + t * 16, 16)
                    ]
                pltpu.sync_copy(ones_v, pres_s.at[sidx_v], add=True)
                return 0

            nstream = (cnt_own + _SCH - 1) // _SCH
            lax.fori_loop(0, nstream, add_chunk, 0)
            return 0

        lax.fori_loop(0, _TBL // _LSEG, half_seg, 0)
        plsc.subcore_barrier()

        pltpu.sync_copy(pres_s.at[pl.ds(sid * _PCHUNK, _PCHUNK)], pres_v)

        def count_body(v, acc):
            return acc + pres_v[pl.ds(v * 16, 16)]

        acc = lax.fori_loop(
            0, _PCHUNK // 16, count_body, jnp.zeros((16,), jnp.int32),
            unroll=4,
        )
        my_cnt = jnp.sum(acc)
        cnt_v[...] = jnp.full((16,), my_cnt, jnp.int32)
        pltpu.sync_copy(cnt_v, cnts_s.at[pl.ds(sid * 16, 16)])
        plsc.subcore_barrier()
        pltpu.sync_copy(cnts_s, cnts_v)
        counts = plsc.load_gather(cnts_v, [iota * 16])
        zero16 = jnp.zeros((16,), jnp.int32)
        prefix = jnp.sum(jnp.where(iota < my_tid, counts, zero16))
        total = jnp.sum(counts)

        def compact_body(v, off):
            pv = pres_v[pl.ds(v * 16, 16)]
            sel = pv > 0
            ivec = sid * _PCHUNK + v * 16 + iota
            plsc.store_compressed(cmp_v.at[pl.ds(off, 16)], ivec, mask=sel)
            return off + jnp.sum(sel.astype(jnp.int32))

        cnt_r = lax.fori_loop(0, _PCHUNK // 16, compact_body, jnp.int32(0))

        emit_n = jnp.clip(jnp.int32(_SAMPLE) - prefix, 0, cnt_r)
        obase = (b * _OUTP + prefix) * 3
        tbase = (b * _OUTP + _SAMPLE) * 3

        def emit_body(j, _c):
            idxs = cmp_v[pl.ds(j * 16, 16)]
            k16 = j * 16 + iota
            vm = k16 < emit_n
            srow = (b * _N + jnp.where(vm, idxs, 0)) * 3
            gidx_v[pl.ds(0, 16)] = srow
            gidx_v[pl.ds(16, 16)] = srow + 1
            gidx_v[pl.ds(32, 16)] = srow + 2
            d1 = pltpu.async_copy(
                xyzf_hbm.at[gidx_v.at[pl.ds(0, 16)]], gx_v, sem_g
            )
            d2 = pltpu.async_copy(
                xyzf_hbm.at[gidx_v.at[pl.ds(16, 16)]], gy_v, sem_g
            )
            d3 = pltpu.async_copy(
                xyzf_hbm.at[gidx_v.at[pl.ds(32, 16)]], gz_v, sem_g
            )
            drow = jnp.where(vm, obase + k16 * 3, tbase)
            ox_v[...] = drow
            oy_v[...] = drow + 1
            oz_v[...] = drow + 2
            d1.wait()
            d2.wait()
            d3.wait()
            e1 = pltpu.async_copy(gx_v, out_hbm.at[ox_v], sem_s)
            e2 = pltpu.async_copy(gy_v, out_hbm.at[oy_v], sem_s)
            e3 = pltpu.async_copy(gz_v, out_hbm.at[oz_v], sem_s)
            e1.wait()
            e2.wait()
            e3.wait()
            return 0

        lax.fori_loop(0, (emit_n + 15) // 16, emit_body, 0)

        fill_lo = jnp.maximum(total, sid * (_SAMPLE // _NSUB))
        fill_hi = (sid + 1) * (_SAMPLE // _NSUB)
        nfill = jnp.maximum(fill_hi - fill_lo, 0)

        def fill_flat(j, _c):
            f0 = (b * _OUTP + fill_lo) * 3 + j * 16
            fpos = f0 + iota
            lim = (b * _OUTP + fill_hi) * 3
            vm = fpos < lim
            coord = (fpos - b * _OUTP * 3) % 3
            gidx_v[pl.ds(0, 16)] = b * _N * 3 + coord
            d = pltpu.async_copy(
                xyzf_hbm.at[gidx_v.at[pl.ds(0, 16)]], gx_v, sem_g
            )
            ox_v[...] = jnp.where(vm, fpos, tbase)
            d.wait()
            e = pltpu.async_copy(gx_v, out_hbm.at[ox_v], sem_s)
            e.wait()
            return 0

        lax.fori_loop(0, (nfill * 3 + 15) // 16, fill_flat, 0)
        return 0

    lax.fori_loop(0, 2, one_batch, 0)


def _sc_sample(lin, xyzf):
    mesh = plsc.VectorSubcoreMesh(
        core_axis_name="c", subcore_axis_name="s"
    )
    f = pl.kernel(
        _sc_body,
        out_type=jax.ShapeDtypeStruct((_B * _OUTP * 3,), jnp.float32),
        mesh=mesh,
        compiler_params=pltpu.CompilerParams(needs_layout_passes=False),
        scratch_types=[
            pltpu.VMEM((_TBL,), jnp.int32),
            pltpu.VMEM((_CHUNK,), jnp.int32),
            pltpu.VMEM((_LSEG + _SCH,), jnp.int32),
            pltpu.VMEM((_SCH,), jnp.int32),
            pltpu.VMEM((_PCHUNK,), jnp.int32),
            pltpu.VMEM((_SAMPLE + 16,), jnp.int32),
            pltpu.VMEM((1024,), jnp.int32),
            pltpu.VMEM((_SCH,), jnp.int32),
            pltpu.VMEM((48,), jnp.int32),
            pltpu.VMEM((16,), jnp.float32),
            pltpu.VMEM((16,), jnp.float32),
            pltpu.VMEM((16,), jnp.float32),
            pltpu.VMEM((16,), jnp.int32),
            pltpu.VMEM((16,), jnp.int32),
            pltpu.VMEM((16,), jnp.int32),
            pltpu.VMEM((16,), jnp.int32),
            pltpu.VMEM((256,), jnp.int32),
            pltpu.VMEM_SHARED((_N,), jnp.int32),
            pltpu.VMEM_SHARED((_N + 16,), jnp.int32),
            pltpu.VMEM_SHARED((256,), jnp.int32),
            pltpu.SemaphoreType.DMA,
            pltpu.SemaphoreType.DMA,
        ],
    )
    return f(lin, xyzf)


def kernel(xyz):
    xyz_t = jnp.transpose(xyz, (0, 2, 1))
    lin = _voxel_ids(xyz_t)
    xyzf = xyz.reshape(-1)
    out = _sc_sample(lin, xyzf)
    return out.reshape(_B, _OUTP, 3)[:, :_SAMPLE, :]


# speedup vs baseline: 1.2153x; 1.2153x over previous
import functools

import jax
import jax.numpy as jnp
from jax import lax
from jax.experimental import pallas as pl
from jax.experimental.pallas import tpu as pltpu
from jax.experimental.pallas import tpu_sc as plsc

_B = 4
_N = 262144
_SAMPLE = 16384
_OUTP = _SAMPLE + 16
_NVOX = 512000
_SENT = 524288
_EMPTY = 0x7FFFFFFF
_TBL = 32768
_NSUB = 16
_NCORE = 2
_CHUNK = 2048
_LSEG = 16384
_PCHUNK = _N // _NSUB
_SCH = 128


def _bin_body(xyz_ref, lin_ref):
    lower = jnp.float32(-4.0)
    voxel = jnp.float32(0.1)
    grid = jnp.float32(80.0)
    x = xyz_ref[0, 0, :]
    y = xyz_ref[0, 1, :]
    z = xyz_ref[0, 2, :]
    cx = jnp.floor((x - lower) / voxel)
    cy = jnp.floor((y - lower) / voxel)
    cz = jnp.floor((z - lower) / voxel)
    valid = (
        (cx >= 0.0) & (cx < grid)
        & (cy >= 0.0) & (cy < grid)
        & (cz >= 0.0) & (cz < grid)
    )
    zero = jnp.float32(0.0)
    ix = jnp.where(valid, cx, zero).astype(jnp.int32)
    iy = jnp.where(valid, cy, zero).astype(jnp.int32)
    iz = jnp.where(valid, cz, zero).astype(jnp.int32)
    lin = ix * 6400 + iy * 80 + iz
    lin_ref[0, 0, :] = jnp.where(valid, lin, jnp.int32(_SENT))


def _voxel_ids(xyz_t):
    return pl.pallas_call(
        _bin_body,
        out_shape=jax.ShapeDtypeStruct((_B, 1, _N), jnp.int32),
        grid=(_B,),
        in_specs=[pl.BlockSpec((1, 3, _N), lambda i: (i, 0, 0))],
        out_specs=pl.BlockSpec((1, 1, _N), lambda i: (i, 0, 0)),
    )(xyz_t).reshape(_B, _N)


def _sc_body(
    lin_hbm,
    xyzf_hbm,
    out_hbm,
    tbl_v,
    buf_v,
    fl_l,
    fl_i,
    fl_l2,
    fl_i2,
    lst_v,
    sidx_v,
    pres_v,
    cmp_v,
    zbuf_v,
    ones_v,
    gidx_v,
    gx_v,
    gy_v,
    gz_v,
    ox_v,
    oy_v,
    oz_v,
    cnt_v,
    cnts_v,
    lin_s,
    pres_s,
    cnts_s,
    sem_g,
    sem_s,
):
    sid = lax.axis_index("s")
    cid = lax.axis_index("c")
    iota = lax.iota(jnp.int32, 16)
    my_tid = jnp.int32(sid)
    prev_idx = jnp.maximum(iota - 1, 0)

    @functools.partial(lax.fori_loop, 0, 1024 // 16, init_val=0)
    def _(i, c):
        zbuf_v[pl.ds(i * 16, 16)] = jnp.zeros((16,), jnp.int32)
        return c

    @functools.partial(lax.fori_loop, 0, _SCH // 16, init_val=0)
    def _(i, c):
        ones_v[pl.ds(i * 16, 16)] = jnp.ones((16,), jnp.int32)
        return c

    def one_batch(step, _carry):
        b = cid * 2 + step

        for j in range(_PCHUNK // 1024):
            pltpu.sync_copy(
                zbuf_v, pres_s.at[pl.ds(sid * _PCHUNK + j * 1024, 1024)]
            )
        pltpu.sync_copy(
            lin_hbm.at[b, pl.ds(sid * _PCHUNK, _PCHUNK)],
            lin_s.at[pl.ds(sid * _PCHUNK, _PCHUNK)],
        )
        @functools.partial(lax.fori_loop, 0, _TBL // 16, init_val=0, unroll=4)
        def _(i, c):
            tbl_v[pl.ds(i * 16, 16)] = jnp.full((16,), _EMPTY, jnp.int32)
            return c

        plsc.subcore_barrier()

        def chunk_body(k, _c):
            pltpu.sync_copy(lin_s.at[pl.ds(k * _CHUNK, _CHUNK)], buf_v)
            base = k * _CHUNK

            def filt(v, offs):
                off_a, off_b = offs
                lin_a = buf_v[pl.ds(v * 32, 16)]
                lin_b = buf_v[pl.ds(v * 32 + 16, 16)]
                own_a = lax.shift_right_logical(lin_a, 15) == my_tid
                own_b = lax.shift_right_logical(lin_b, 15) == my_tid
                plsc.store_compressed(
                    fl_l.at[pl.ds(off_a, 16)],
                    lin_a & jnp.int32(_TBL - 1),
                    mask=own_a,
                )
                plsc.store_compressed(
                    fl_l2.at[pl.ds(off_b, 16)],
                    lin_b & jnp.int32(_TBL - 1),
                    mask=own_b,
                )
                plsc.store_compressed(
                    fl_i.at[pl.ds(off_a, 16)],
                    base + v * 32 + iota,
                    mask=own_a,
                )
                plsc.store_compressed(
                    fl_i2.at[pl.ds(off_b, 16)],
                    base + v * 32 + 16 + iota,
                    mask=own_b,
                )
                return (
                    off_a + jnp.sum(own_a.astype(jnp.int32)),
                    off_b + jnp.sum(own_b.astype(jnp.int32)),
                )

            cnt_a, cnt_b = lax.fori_loop(
                0, _CHUNK // 32, filt, (jnp.int32(0), jnp.int32(0)),
                unroll=4,
            )
            fl_l[pl.ds(cnt_a, 16)] = jnp.zeros((16,), jnp.int32)
            fl_i[pl.ds(cnt_a, 16)] = jnp.full((16,), _EMPTY, jnp.int32)
            fl_l2[pl.ds(cnt_b, 16)] = jnp.zeros((16,), jnp.int32)
            fl_i2[pl.ds(cnt_b, 16)] = jnp.full((16,), _EMPTY, jnp.int32)

            def make_upd(l_ref, i_ref):
                def upd(v, _c2):
                    lidx = l_ref[pl.ds(v * 16, 16)]
                    ivec = i_ref[pl.ds(v * 16, 16)]
                    cur = plsc.load_gather(tbl_v, [lidx])
                    pending = ivec < cur

                    def fix_body(p):
                        plsc.store_scatter(tbl_v, [lidx], ivec, mask=p)
                        got = plsc.load_gather(tbl_v, [lidx], mask=p)
                        return p & (ivec < got)

                    lax.while_loop(
                        lambda p: jnp.sum(p.astype(jnp.int32)) > 0,
                        fix_body,
                        pending,
                    )
                    return 0

                return upd

            lax.fori_loop(0, (cnt_a + 15) // 16, make_upd(fl_l, fl_i), 0)
            lax.fori_loop(0, (cnt_b + 15) // 16, make_upd(fl_l2, fl_i2), 0)
            return 0

        lax.fori_loop(0, _N // _CHUNK, chunk_body, 0)

        def half_seg(seg, _c0):
            def scan_tbl(v, off):
                vals = tbl_v[pl.ds(seg * _LSEG + v * 16, 16)]
                sel = vals != _EMPTY
                plsc.store_compressed(
                    lst_v.at[pl.ds(off, 16)], vals, mask=sel
                )
                return off + jnp.sum(sel.astype(jnp.int32))

            cnt_own = lax.fori_loop(0, _LSEG // 16, scan_tbl, jnp.int32(0))

            @functools.partial(lax.fori_loop, 0, _SCH // 16, init_val=0)
            def _(i, c):
                lst_v[pl.ds(cnt_own + i * 16, 16)] = jnp.full(
                    (16,), _N, jnp.int32
                )
                return c

            def add_chunk(k, _c):
                for t in range(_SCH // 16):
                    sidx_v[pl.ds(t * 16, 16)] = lst_v[
                        pl.ds(k * _SCH + t * 16, 16)
                    ]
                pltpu.sync_copy(ones_v, pres_s.at[sidx_v], add=True)
                return 0

            nstream = (cnt_own + _SCH - 1) // _SCH
            lax.fori_loop(0, nstream, add_chunk, 0)
            return 0

        lax.fori_loop(0, _TBL // _LSEG, half_seg, 0)
        plsc.subcore_barrier()

        pltpu.sync_copy(pres_s.at[pl.ds(sid * _PCHUNK, _PCHUNK)], pres_v)

        def count_body(v, acc):
            return acc + pres_v[pl.ds(v * 16, 16)]

        acc = lax.fori_loop(
            0, _PCHUNK // 16, count_body, jnp.zeros((16,), jnp.int32),
            unroll=4,
        )
        my_cnt = jnp.sum(acc)
        cnt_v[...] = jnp.full((16,), my_cnt, jnp.int32)
        pltpu.sync_copy(cnt_v, cnts_s.at[pl.ds(sid * 16, 16)])
        plsc.subcore_barrier()
        pltpu.sync_copy(cnts_s, cnts_v)
        counts = plsc.load_gather(cnts_v, [iota * 16])
        zero16 = jnp.zeros((16,), jnp.int32)
        prefix = jnp.sum(jnp.where(iota < my_tid, counts, zero16))
        total = jnp.sum(counts)

        def compact_body(v, off):
            pv = pres_v[pl.ds(v * 16, 16)]
            sel = pv > 0
            ivec = sid * _PCHUNK + v * 16 + iota
            plsc.store_compressed(cmp_v.at[pl.ds(off, 16)], ivec, mask=sel)
            return off + jnp.sum(sel.astype(jnp.int32))

        cnt_r = lax.fori_loop(0, _PCHUNK // 16, compact_body, jnp.int32(0))

        emit_n = jnp.clip(jnp.int32(_SAMPLE) - prefix, 0, cnt_r)
        obase = (b * _OUTP + prefix) * 3
        tbase = (b * _OUTP + _SAMPLE) * 3

        def emit_body(j, _c):
            for t in range(8):
                idxs = cmp_v[pl.ds(j * 128 + t * 16, 16)]
                k16 = j * 128 + t * 16 + iota
                vm = k16 < emit_n
                srow = (b * _N + jnp.where(vm, idxs, 0)) * 3
                gidx_v[pl.ds(t * 16, 16)] = srow
                gidx_v[pl.ds(128 + t * 16, 16)] = srow + 1
                gidx_v[pl.ds(256 + t * 16, 16)] = srow + 2
                drow = jnp.where(vm, obase + k16 * 3, tbase)
                ox_v[pl.ds(t * 16, 16)] = drow
                oy_v[pl.ds(t * 16, 16)] = drow + 1
                oz_v[pl.ds(t * 16, 16)] = drow + 2
            d1 = pltpu.async_copy(
                xyzf_hbm.at[gidx_v.at[pl.ds(0, 128)]], gx_v, sem_g
            )
            d2 = pltpu.async_copy(
                xyzf_hbm.at[gidx_v.at[pl.ds(128, 128)]], gy_v, sem_g
            )
            d3 = pltpu.async_copy(
                xyzf_hbm.at[gidx_v.at[pl.ds(256, 128)]], gz_v, sem_g
            )
            d1.wait()
            d2.wait()
            d3.wait()
            e1 = pltpu.async_copy(gx_v, out_hbm.at[ox_v], sem_s)
            e2 = pltpu.async_copy(gy_v, out_hbm.at[oy_v], sem_s)
            e3 = pltpu.async_copy(gz_v, out_hbm.at[oz_v], sem_s)
            e1.wait()
            e2.wait()
            e3.wait()
            return 0

        lax.fori_loop(0, (emit_n + 127) // 128, emit_body, 0)

        fill_lo = jnp.maximum(total, sid * (_SAMPLE // _NSUB))
        fill_hi = (sid + 1) * (_SAMPLE // _NSUB)
        nfill = jnp.maximum(fill_hi - fill_lo, 0)

        def fill_flat(j, _c):
            lim = (b * _OUTP + fill_hi) * 3
            for t in range(8):
                f0 = (b * _OUTP + fill_lo) * 3 + j * 128 + t * 16
                fpos = f0 + iota
                vm = fpos < lim
                coord = (fpos - b * _OUTP * 3) % 3
                gidx_v[pl.ds(t * 16, 16)] = b * _N * 3 + coord
                ox_v[pl.ds(t * 16, 16)] = jnp.where(vm, fpos, tbase)
            d = pltpu.async_copy(
                xyzf_hbm.at[gidx_v.at[pl.ds(0, 128)]], gx_v, sem_g
            )
            d.wait()
            e = pltpu.async_copy(gx_v, out_hbm.at[ox_v], sem_s)
            e.wait()
            return 0

        lax.fori_loop(0, (nfill * 3 + 127) // 128, fill_flat, 0)
        return 0

    lax.fori_loop(0, 2, one_batch, 0)


def _sc_sample(lin, xyzf):
    mesh = plsc.VectorSubcoreMesh(
        core_axis_name="c", subcore_axis_name="s"
    )
    f = pl.kernel(
        _sc_body,
        out_type=jax.ShapeDtypeStruct((_B * _OUTP * 3,), jnp.float32),
        mesh=mesh,
        compiler_params=pltpu.CompilerParams(needs_layout_passes=False),
        scratch_types=[
            pltpu.VMEM((_TBL,), jnp.int32),
            pltpu.VMEM((_CHUNK,), jnp.int32),
            pltpu.VMEM((_CHUNK // 2 + 16,), jnp.int32),
            pltpu.VMEM((_CHUNK // 2 + 16,), jnp.int32),
            pltpu.VMEM((_CHUNK // 2 + 16,), jnp.int32),
            pltpu.VMEM((_CHUNK // 2 + 16,), jnp.int32),
            pltpu.VMEM((_LSEG + _SCH,), jnp.int32),
            pltpu.VMEM((_SCH,), jnp.int32),
            pltpu.VMEM((_PCHUNK,), jnp.int32),
            pltpu.VMEM((_SAMPLE + 16,), jnp.int32),
            pltpu.VMEM((1024,), jnp.int32),
            pltpu.VMEM((_SCH,), jnp.int32),
            pltpu.VMEM((384,), jnp.int32),
            pltpu.VMEM((128,), jnp.float32),
            pltpu.VMEM((128,), jnp.float32),
            pltpu.VMEM((128,), jnp.float32),
            pltpu.VMEM((128,), jnp.int32),
            pltpu.VMEM((128,), jnp.int32),
            pltpu.VMEM((128,), jnp.int32),
            pltpu.VMEM((16,), jnp.int32),
            pltpu.VMEM((256,), jnp.int32),
            pltpu.VMEM_SHARED((_N,), jnp.int32),
            pltpu.VMEM_SHARED((_N + 16,), jnp.int32),
            pltpu.VMEM_SHARED((256,), jnp.int32),
            pltpu.SemaphoreType.DMA,
            pltpu.SemaphoreType.DMA,
        ],
    )
    return f(lin, xyzf)


def kernel(xyz):
    xyz_t = jnp.transpose(xyz, (0, 2, 1))
    lin = _voxel_ids(xyz_t)
    xyzf = xyz.reshape(-1)
    out = _sc_sample(lin, xyzf)
    return out.reshape(_B, _OUTP, 3)[:, :_SAMPLE, :]
